# baseline (speedup 1.0000x reference)
import jax
import jax.numpy as jnp
from jax.experimental import pallas as pl

_HOP = 256
_SEG = 512


def _frames_kernel(x_ref, w_ref, o_ref):
    nf = o_ref.shape[1]
    h = pl.program_id(1)

    @pl.when(h == 0)
    def _first():
        o_ref[0, :, :] = x_ref[0, 0:nf, :] * w_ref[0, :]

    @pl.when(h == 1)
    def _second():
        o_ref[0, :, :] = x_ref[0, 1:nf + 1, :] * w_ref[1, :]


def kernel(x, analysis_window):
    batch, num_samples = x.shape
    num_chunks = num_samples // _HOP
    num_frames = (num_samples - _SEG) // _HOP + 1

    x3 = x.reshape(batch, num_chunks, _HOP)
    w2 = analysis_window.reshape(2, _HOP)

    return pl.pallas_call(
        _frames_kernel,
        grid=(batch, 2),
        in_specs=[
            pl.BlockSpec((1, num_chunks, _HOP), lambda b, h: (b, 0, 0)),
            pl.BlockSpec((2, _HOP), lambda b, h: (0, 0)),
        ],
        out_specs=pl.BlockSpec((1, num_frames, _HOP), lambda b, h: (b, 0, h)),
        out_shape=jax.ShapeDtypeStruct((batch, num_frames, _SEG), x.dtype),
    )(x3, w2)


# speedup vs baseline: 1.0708x; 1.0708x over previous
import jax
import jax.numpy as jnp
from jax.experimental import pallas as pl
from jax.experimental.pallas import tpu as pltpu

_HOP = 256
_SEG = 512
_NQ = 4


def _frames_kernel(x_ref, w_ref, o_hbm, scratch, sems):
    b = pl.program_id(0)
    nb = pl.num_programs(0)
    nf = o_hbm.shape[1]
    buf = b % 2
    tile = (nf + _NQ - 1) // _NQ
    tile = (tile + 7) // 8 * 8
    bounds = [(q * tile, min((q + 1) * tile, nf)) for q in range(_NQ)]

    def q_copy(bb, qq, row):
        lo, hi = bounds[qq]
        return pltpu.make_async_copy(
            scratch.at[bb, pl.ds(lo, hi - lo), :],
            o_hbm.at[row, pl.ds(lo, hi - lo), :],
            sems.at[bb, qq],
        )

    @pl.when(b >= 2)
    def _drain_prev():
        for q in range(_NQ):
            q_copy(buf, q, b - 2).wait()

    a = x_ref[0, 0:nf, :]
    c = x_ref[0, 1:nf + 1, :]
    scratch[buf, :, 0:_HOP] = a * w_ref[0, :]
    scratch[buf, :, _HOP:_SEG] = c * w_ref[1, :]

    for q in range(_NQ):
        q_copy(buf, q, b).start()

    @pl.when(b == nb - 1)
    def _drain_tail():
        for q in range(_NQ):
            q_copy(1 - buf, q, b - 1).wait()
        for q in range(_NQ):
            q_copy(buf, q, b).wait()


def kernel(x, analysis_window):
    batch, num_samples = x.shape
    num_chunks = num_samples // _HOP
    num_frames = (num_samples - _SEG) // _HOP + 1

    x3 = x.reshape(batch, num_chunks, _HOP)
    w2 = analysis_window.reshape(2, _HOP)

    return pl.pallas_call(
        _frames_kernel,
        grid=(batch,),
        in_specs=[
            pl.BlockSpec((1, num_chunks, _HOP), lambda b: (b, 0, 0)),
            pl.BlockSpec((2, _HOP), lambda b: (0, 0)),
        ],
        out_specs=pl.BlockSpec(memory_space=pltpu.MemorySpace.HBM),
        out_shape=jax.ShapeDtypeStruct((batch, num_frames, _SEG), x.dtype),
        scratch_shapes=[
            pltpu.VMEM((2, num_frames, _SEG), x.dtype),
            pltpu.SemaphoreType.DMA((2, _NQ)),
        ],
    )(x3, w2)


# speedup vs baseline: 1.1313x; 1.0565x over previous
import jax
import jax.numpy as jnp
from jax.experimental import pallas as pl
from jax.experimental.pallas import tpu as pltpu

_HOP = 256
_SEG = 512
_KT = 512
_NT = 8
_MAIN = 4088
_TAIL = 7


def _frames_kernel(x_ref, w_ref, o_hbm, tail_ref, scratch, sems):
    b = pl.program_id(0)
    nb = pl.num_programs(0)
    w0 = w_ref[0, :]
    w1 = w_ref[1, :]

    starts = [t * _KT for t in range(_NT)]
    sizes = [min(_KT, _MAIN - t * _KT) for t in range(_NT)]

    def t_copy(t, row):
        return pltpu.make_async_copy(
            scratch.at[t, pl.ds(0, sizes[t]), :],
            o_hbm.at[row, pl.ds(starts[t], sizes[t]), :],
            sems.at[t],
        )

    for t in range(_NT):
        k0, sz = starts[t], sizes[t]

        @pl.when(b >= 1)
        def _drain_prev(t=t):
            t_copy(t, b - 1).wait()

        scratch[t, 0:sz, 0:_HOP] = x_ref[0, k0:k0 + sz, :] * w0
        scratch[t, 0:sz, _HOP:_SEG] = x_ref[0, k0 + 1:k0 + sz + 1, :] * w1
        t_copy(t, b).start()

    tail_ref[0, :, 0:_HOP] = x_ref[0, _MAIN:_MAIN + _TAIL, :] * w0
    tail_ref[0, :, _HOP:_SEG] = x_ref[0, _MAIN + 1:_MAIN + _TAIL + 1, :] * w1

    @pl.when(b == nb - 1)
    def _drain_tail():
        for t in range(_NT):
            t_copy(t, b).wait()


def kernel(x, analysis_window):
    batch, num_samples = x.shape
    num_chunks = num_samples // _HOP
    num_frames = (num_samples - _SEG) // _HOP + 1

    x3 = x.reshape(batch, num_chunks, _HOP)
    w2 = analysis_window.reshape(2, _HOP)

    main, tail = pl.pallas_call(
        _frames_kernel,
        grid=(batch,),
        in_specs=[
            pl.BlockSpec((1, num_chunks, _HOP), lambda b: (b, 0, 0)),
            pl.BlockSpec((2, _HOP), lambda b: (0, 0)),
        ],
        out_specs=[
            pl.BlockSpec(memory_space=pltpu.MemorySpace.HBM),
            pl.BlockSpec((1, _TAIL, _SEG), lambda b: (b, 0, 0)),
        ],
        out_shape=[
            jax.ShapeDtypeStruct((batch, num_frames, _SEG), x.dtype),
            jax.ShapeDtypeStruct((batch, _TAIL, _SEG), x.dtype),
        ],
        scratch_shapes=[
            pltpu.VMEM((_NT, _KT, _SEG), x.dtype),
            pltpu.SemaphoreType.DMA((_NT,)),
        ],
    )(x3, w2)
    return jax.lax.dynamic_update_slice(main, tail, (0, _MAIN, 0))
